# baseline (speedup 1.0000x reference)
import functools

import jax
import jax.numpy as jnp
from jax import lax
from jax.experimental import pallas as pl
from jax.experimental.pallas import tpu as pltpu
from jax.experimental.pallas import tpu_sc as plsc

DIM = 64
BATCH = 4
SEQ = 8192
NTOK = BATCH * SEQ
NW = 32
BPW = NTOK // NW
CH = 256
NCH = BPW // CH

_mesh = plsc.VectorSubcoreMesh(core_axis_name="c", subcore_axis_name="s")


@functools.partial(
    pl.kernel,
    mesh=_mesh,
    out_type=jax.ShapeDtypeStruct((NTOK, DIM), jnp.float32),
    compiler_params=pltpu.CompilerParams(use_tc_tiling_on_sc=False),
    scratch_types=[
        pltpu.VMEM((BPW,), jnp.int32),
        pltpu.VMEM((BPW, DIM), jnp.float32),
        pltpu.VMEM((CH, DIM), jnp.float32),
        pltpu.SemaphoreType.DMA,
    ],
)
def _embed(ids_hbm, tok_hbm, pos_hbm, out_hbm, idx_v, pos_v, rows_v, sem):
    wid = lax.axis_index("s") * 2 + lax.axis_index("c")
    base = wid * BPW
    pltpu.sync_copy(ids_hbm.at[pl.ds(base, BPW)], idx_v)
    pos_base = lax.rem(base, SEQ)
    pltpu.sync_copy(pos_hbm.at[pl.ds(pos_base, BPW)], pos_v)
    for c in range(NCH):
        pltpu.async_copy(
            tok_hbm.at[idx_v.at[pl.ds(c * CH, CH)]], rows_v, sem
        ).wait()

        def body(i, _, c=c):
            for j in range(DIM // 16):
                sl = pl.ds(j * 16, 16)
                rows_v[i, sl] = rows_v[i, sl] + pos_v[c * CH + i, sl]
            return 0

        lax.fori_loop(0, CH, body, 0)
        pltpu.sync_copy(rows_v, out_hbm.at[pl.ds(base + c * CH, CH)])


def kernel(input_ids, token_table, pos_table):
    ids = input_ids.reshape(NTOK).astype(jnp.int32)
    out = _embed(ids, token_table, pos_table)
    return out.reshape(BATCH, SEQ, DIM)


# speedup vs baseline: 1.0188x; 1.0188x over previous
import functools

import jax
import jax.numpy as jnp
from jax import lax
from jax.experimental import pallas as pl
from jax.experimental.pallas import tpu as pltpu
from jax.experimental.pallas import tpu_sc as plsc

DIM = 64
BATCH = 4
SEQ = 8192
NTOK = BATCH * SEQ
NW = 32
BPW = NTOK // NW
CH = 256
NCH = BPW // CH

_mesh = plsc.VectorSubcoreMesh(core_axis_name="c", subcore_axis_name="s")


@functools.partial(
    pl.kernel,
    mesh=_mesh,
    out_type=jax.ShapeDtypeStruct((NTOK, DIM), jnp.float32),
    compiler_params=pltpu.CompilerParams(use_tc_tiling_on_sc=False),
    scratch_types=[
        pltpu.VMEM((BPW,), jnp.int32),
        pltpu.VMEM((CH, DIM), jnp.float32),
        pltpu.SemaphoreType.DMA,
    ],
)
def _embed(ids_hbm, tok_hbm, pos_hbm, out_hbm, idx_v, rows_v, sem):
    wid = lax.axis_index("s") * 2 + lax.axis_index("c")
    base = wid * BPW
    pltpu.sync_copy(ids_hbm.at[pl.ds(base, BPW)], idx_v)
    pos_base = lax.rem(base, SEQ)
    for c in range(NCH):
        pltpu.sync_copy(pos_hbm.at[pl.ds(pos_base + c * CH, CH)], rows_v)
        pltpu.async_copy(
            tok_hbm.at[idx_v.at[pl.ds(c * CH, CH)]], rows_v, sem, add=True
        ).wait()
        pltpu.sync_copy(rows_v, out_hbm.at[pl.ds(base + c * CH, CH)])


def kernel(input_ids, token_table, pos_table):
    ids = input_ids.reshape(NTOK).astype(jnp.int32)
    out = _embed(ids, token_table, pos_table)
    return out.reshape(BATCH, SEQ, DIM)


# speedup vs baseline: 1.0627x; 1.0430x over previous
import functools

import jax
import jax.numpy as jnp
from jax import lax
from jax.experimental import pallas as pl
from jax.experimental.pallas import tpu as pltpu
from jax.experimental.pallas import tpu_sc as plsc

DIM = 64
BATCH = 4
SEQ = 8192
NTOK = BATCH * SEQ
NW = 32
BPW = NTOK // NW
NB = 4
CH = BPW // NB
NCH = BPW // CH

_mesh = plsc.VectorSubcoreMesh(core_axis_name="c", subcore_axis_name="s")


@functools.partial(
    pl.kernel,
    mesh=_mesh,
    out_type=jax.ShapeDtypeStruct((NTOK, DIM), jnp.float32),
    compiler_params=pltpu.CompilerParams(use_tc_tiling_on_sc=False),
    scratch_types=(
        [pltpu.VMEM((BPW,), jnp.int32)]
        + [pltpu.VMEM((CH, DIM), jnp.float32) for _ in range(NB)]
        + [pltpu.SemaphoreType.DMA for _ in range(3 * NB)]
    ),
)
def _embed(ids_hbm, tok_hbm, pos_hbm, out_hbm, idx_v, *bufsems):
    rows = bufsems[:NB]
    sem_pos = bufsems[NB:2 * NB]
    sem_tok = bufsems[2 * NB:3 * NB]
    sem_out = bufsems[3 * NB:]
    wid = lax.axis_index("s") * 2 + lax.axis_index("c")
    base = wid * BPW
    pltpu.sync_copy(ids_hbm.at[pl.ds(base, BPW)], idx_v)
    pos_base = lax.rem(base, SEQ)

    pos_dma = [None] * NCH
    tok_dma = [None] * NCH
    out_dma = [None] * NCH
    for c in range(NCH):
        pos_dma[c] = pltpu.async_copy(
            pos_hbm.at[pl.ds(pos_base + c * CH, CH)], rows[c % NB],
            sem_pos[c % NB])
    for c in range(NCH):
        pos_dma[c].wait()
        tok_dma[c] = pltpu.async_copy(
            tok_hbm.at[idx_v.at[pl.ds(c * CH, CH)]], rows[c % NB],
            sem_tok[c % NB], add=True)
    for c in range(NCH):
        tok_dma[c].wait()
        out_dma[c] = pltpu.async_copy(
            rows[c % NB], out_hbm.at[pl.ds(base + c * CH, CH)],
            sem_out[c % NB])
    for c in range(NCH):
        out_dma[c].wait()


def kernel(input_ids, token_table, pos_table):
    ids = input_ids.reshape(NTOK).astype(jnp.int32)
    out = _embed(ids, token_table, pos_table)
    return out.reshape(BATCH, SEQ, DIM)
